# baseline (speedup 1.0000x reference)
import functools

import jax
import jax.numpy as jnp
from jax import lax
from jax.experimental import pallas as pl
from jax.experimental.pallas import tpu as pltpu
from jax.experimental.pallas import tpu_sc as plsc

N_NODES = 10000
D = 128
DE = 16
E = 320000
TWO_E = 2 * E

NC, NS = 2, 16
NW = NC * NS
PAD_N = 10240

_mesh = plsc.VectorSubcoreMesh(core_axis_name="c", subcore_axis_name="s")


def _wid():
    return lax.axis_index("s") * NC + lax.axis_index("c")


CH2 = TWO_E // NW
NCH2 = CH2 // 16


@functools.partial(
    pl.kernel,
    out_type=jax.ShapeDtypeStruct((NW * PAD_N,), jnp.int32),
    mesh=_mesh,
    compiler_params=pltpu.CompilerParams(needs_layout_passes=False),
    scratch_types=[pltpu.VMEM((CH2,), jnp.int32), pltpu.VMEM((PAD_N,), jnp.int32)],
)
def _winner_partial(inv_hbm, parts_hbm, idx_v, wtab):
    wid = _wid()
    pltpu.sync_copy(inv_hbm.at[pl.ds(wid * CH2, CH2)], idx_v)
    neg1 = jnp.full((16,), -1, jnp.int32)

    def initb(i, carry):
        wtab[pl.ds(i * 16, 16)] = neg1
        return carry

    lax.fori_loop(0, PAD_N // 16, initb, 0)

    lanes = lax.iota(jnp.int32, 16)
    shift1 = jnp.minimum(lanes + 1, 15)
    dnums = lax.GatherDimensionNumbers(
        offset_dims=(), collapsed_slice_dims=(0,), start_index_map=(0,))
    last_lane = lanes == 15
    base_j = wid * CH2

    def body(k, carry):
        idx16 = idx_v[pl.ds(k * 16, 16)]
        j16 = base_j + k * 16 + lanes
        key = idx16 * 16 + lanes
        sk, sv = plsc.sort_key_val(key, j16)
        sidx = lax.shift_right_logical(sk, 4)
        nxt = lax.gather(sidx, shift1[:, None], dnums, (1,),
                         mode=lax.GatherScatterMode.PROMISE_IN_BOUNDS)
        winner = (sidx != nxt) | last_lane
        plsc.store_scatter(wtab, [sidx], sv, mask=winner)
        return carry

    lax.fori_loop(0, NCH2, body, 0)
    pltpu.sync_copy(wtab, parts_hbm.at[pl.ds(wid * PAD_N, PAD_N)])


MN = PAD_N // NW


@functools.partial(
    pl.kernel,
    out_type=jax.ShapeDtypeStruct((PAD_N,), jnp.int32),
    mesh=_mesh,
    compiler_params=pltpu.CompilerParams(needs_layout_passes=False),
    scratch_types=[pltpu.VMEM((NW * MN,), jnp.int32), pltpu.VMEM((MN,), jnp.int32)],
)
def _winner_merge(parts_hbm, w_hbm, pv, wout):
    wid = _wid()
    n0 = wid * MN
    for tt in range(NW):
        pltpu.sync_copy(parts_hbm.at[pl.ds(tt * PAD_N + n0, MN)],
                        pv.at[pl.ds(tt * MN, MN)])
    for v in range(MN // 16):
        m = jnp.full((16,), -1, jnp.int32)
        for tt in range(NW):
            m = jnp.maximum(m, pv[pl.ds(tt * MN + v * 16, 16)])
        wout[pl.ds(v * 16, 16)] = m
    pltpu.sync_copy(wout, w_hbm.at[pl.ds(n0, MN)])


RT = PAD_N // NW
NCK, CKS = 4, 80


@functools.partial(
    pl.kernel,
    out_type=(
        jax.ShapeDtypeStruct((PAD_N, D), jnp.float32),
        jax.ShapeDtypeStruct((PAD_N, D), jnp.float32),
        jax.ShapeDtypeStruct((PAD_N, D), jnp.float32),
    ),
    mesh=_mesh,
    compiler_params=pltpu.CompilerParams(needs_layout_passes=False),
    scratch_types=[
        pltpu.VMEM((RT,), jnp.int32),
        pltpu.VMEM((RT, D), jnp.float32),
        pltpu.VMEM((RT, D), jnp.float32),
        pltpu.VMEM((RT, D), jnp.float32),
        pltpu.SemaphoreType.DMA,
    ],
)
def _gather_winners(w_hbm, xs_hbm, xd_hbm, ef_hbm, fa_hbm, fb_hbm, efw_hbm,
                    wv, ra, rb, rc, sem):
    wid = _wid()
    n0 = wid * RT
    pltpu.sync_copy(w_hbm.at[pl.ds(n0, RT)], wv)
    zero = jnp.zeros((16,), jnp.int32)
    copies = []
    for k in range(RT // 16):
        wk = wv[pl.ds(k * 16, 16)]
        is_src = wk < E
        wmod = jnp.where(is_src, wk, wk - E)
        wmod = jnp.maximum(wmod, zero)
        copies.append(pltpu.async_copy(
            xs_hbm.at[jnp.where(is_src, wmod, zero)], ra.at[pl.ds(k * 16, 16)], sem))
        copies.append(pltpu.async_copy(
            xd_hbm.at[jnp.where(is_src, zero, wmod)], rb.at[pl.ds(k * 16, 16)], sem))
        copies.append(pltpu.async_copy(
            ef_hbm.at[lax.shift_right_logical(wmod, 3)], rc.at[pl.ds(k * 16, 16)], sem))
    for cp in copies:
        cp.wait()
    pltpu.sync_copy(ra, fa_hbm.at[pl.ds(n0, RT)])
    pltpu.sync_copy(rb, fb_hbm.at[pl.ds(n0, RT)])
    pltpu.sync_copy(rc, efw_hbm.at[pl.ds(n0, RT)])


EPT = E // NW
HB = 80
NHC = EPT // HB
APC = HB // 16
NPS = PAD_N // NS


@functools.partial(
    pl.kernel,
    out_type=jax.ShapeDtypeStruct((2 * PAD_N, D), jnp.float32),
    mesh=_mesh,
    compiler_params=pltpu.CompilerParams(needs_layout_passes=False),
    scratch_types=[
        pltpu.VMEM((EPT,), jnp.int32),
        pltpu.VMEM((HB, D), jnp.float32),
        pltpu.VMEM((HB, D), jnp.float32),
        pltpu.VMEM_SHARED((PAD_N, D), jnp.float32),
        pltpu.SemaphoreType.DMA,
        pltpu.SemaphoreType.DMA,
        pltpu.SemaphoreType.DMA,
    ],
)
def _segsum(h_hbm, src_hbm, s_hbm, idx_all, st0, st1, s_sh, semh0, semh1, sema):
    core = lax.axis_index("c")
    sid = lax.axis_index("s")
    wid = sid * NC + core
    e_base = wid * EPT
    sts = (st0, st1)
    semhs = (semh0, semh1)
    pltpu.sync_copy(src_hbm.at[pl.ds(e_base, EPT)], idx_all)

    zero16 = jnp.zeros((16,), jnp.float32)

    def zrow(r, carry):
        for q in range(D // 16):
            st0[r, pl.ds(q * 16, 16)] = zero16
        return carry

    lax.fori_loop(0, HB, zrow, 0)
    for q in range(NPS // HB):
        pltpu.sync_copy(st0, s_sh.at[pl.ds(sid * NPS + q * HB, HB)])
    plsc.subcore_barrier()

    def h_chunk_copy(c, b):
        return pltpu.async_copy(h_hbm.at[pl.ds(e_base + c * HB, HB)], sts[b], semhs[b])

    h_chunk_copy(0, 0)
    h_chunk_copy(1, 1)

    def do_chunk(c, b):
        pltpu.make_async_copy(h_hbm.at[pl.ds(e_base + c * HB, HB)], sts[b], semhs[b]).wait()
        descs = []
        for u in range(APC):
            idx16 = idx_all[pl.ds(c * HB + u * 16, 16)]
            descs.append(pltpu.async_copy(
                sts[b].at[pl.ds(u * 16, 16)], s_sh.at[idx16], sema, add=True))
        for dd in descs:
            dd.wait()

    def pair(g, carry):
        for b in range(2):
            c = 2 * g + b
            do_chunk(c, b)

            @pl.when(c + 2 < NHC)
            def _():
                h_chunk_copy(c + 2, b)
        return carry

    lax.fori_loop(0, NHC // 2, pair, 0)
    if NHC % 2:
        do_chunk(NHC - 1, 0)
    plsc.subcore_barrier()
    row0 = core * PAD_N + sid * NPS
    for q in range(NPS // HB):
        pltpu.sync_copy(s_sh.at[pl.ds(sid * NPS + q * HB, HB)],
                        s_hbm.at[pl.ds(row0 + q * HB, HB)])


BE = 2000


def _j1_body(xs_ref, xd_ref, ef_ref, wn_ref, b_ref, we_ref, base_ref, h_ref):
    ep = jnp.dot(ef_ref[...], we_ref[...], preferred_element_type=jnp.float32) + b_ref[...]
    hs = jnp.maximum(jnp.dot(xs_ref[...], wn_ref[...], preferred_element_type=jnp.float32) + ep, 0.0)
    hd = jnp.maximum(jnp.dot(xd_ref[...], wn_ref[...], preferred_element_type=jnp.float32) + ep, 0.0)
    h_ref[...] = hs
    d = jnp.sum(hs * hd, axis=1)
    part = jnp.sum(d * d).reshape(1, 1)

    @pl.when(pl.program_id(0) == 0)
    def _():
        base_ref[...] = part

    @pl.when(pl.program_id(0) != 0)
    def _():
        base_ref[...] = base_ref[...] + part


def _j1_call(x_src, x_dst, edge_feats, W_enc, b2, W_edge):
    return pl.pallas_call(
        _j1_body,
        grid=(E // BE,),
        in_specs=[
            pl.BlockSpec((BE, D), lambda i: (i, 0)),
            pl.BlockSpec((BE, D), lambda i: (i, 0)),
            pl.BlockSpec((BE, DE), lambda i: (i, 0)),
            pl.BlockSpec((D, D), lambda i: (0, 0)),
            pl.BlockSpec((1, D), lambda i: (0, 0)),
            pl.BlockSpec((DE, D), lambda i: (0, 0)),
        ],
        out_specs=[
            pl.BlockSpec((1, 1), lambda i: (0, 0)),
            pl.BlockSpec((BE, D), lambda i: (i, 0)),
        ],
        out_shape=[
            jax.ShapeDtypeStruct((1, 1), jnp.float32),
            jax.ShapeDtypeStruct((E, D), jnp.float32),
        ],
    )(x_src, x_dst, edge_feats, W_enc, b2, W_edge)


BN = 256


def _j4_body(fa_ref, fb_ref, ef_ref, w_ref, wn_ref, b_ref, we_ref, st_ref):
    w = w_ref[...]
    feats = jnp.where(w < E, fa_ref[...], fb_ref[...])
    wmod = jnp.maximum(jnp.where(w < E, w, w - E), 0)
    p = jnp.bitwise_and(wmod, 7)
    efp = ef_ref[...]
    ef16 = jnp.zeros((BN, DE), jnp.float32)
    for pp in range(8):
        ef16 = ef16 + jnp.where(p == pp, efp[:, pp * DE:(pp + 1) * DE], 0.0)
    h = jnp.dot(feats, wn_ref[...], preferred_element_type=jnp.float32)
    h = h + jnp.dot(ef16, we_ref[...], preferred_element_type=jnp.float32) + b_ref[...]
    h = jnp.maximum(h, 0.0)
    st_ref[...] = jnp.where(w >= 0, h, 0.0)


def _j4_call(fa, fb, efw, w2, W_enc, b2, W_edge):
    return pl.pallas_call(
        _j4_body,
        grid=(PAD_N // BN,),
        in_specs=[
            pl.BlockSpec((BN, D), lambda i: (i, 0)),
            pl.BlockSpec((BN, D), lambda i: (i, 0)),
            pl.BlockSpec((BN, D), lambda i: (i, 0)),
            pl.BlockSpec((BN, 1), lambda i: (i, 0)),
            pl.BlockSpec((D, D), lambda i: (0, 0)),
            pl.BlockSpec((1, D), lambda i: (0, 0)),
            pl.BlockSpec((DE, D), lambda i: (0, 0)),
        ],
        out_specs=pl.BlockSpec((BN, D), lambda i: (i, 0)),
        out_shape=jax.ShapeDtypeStruct((PAD_N, D), jnp.float32),
    )(fa, fb, efw, w2, W_enc, b2, W_edge)


def _j6_body(s0_ref, s1_ref, st_ref, base_ref, o_ref):
    inv_e = 1.0 / E
    val = jnp.sum((s0_ref[...] + s1_ref[...]) * st_ref[...]).reshape(1, 1)

    @pl.when(pl.program_id(0) == 0)
    def _():
        o_ref[...] = base_ref[...] * inv_e + 0.001 * inv_e * val

    @pl.when(pl.program_id(0) != 0)
    def _():
        o_ref[...] = o_ref[...] + 0.001 * inv_e * val


def _j6_call(s_big, storage, base_pp):
    nb = PAD_N // BN
    return pl.pallas_call(
        _j6_body,
        grid=(nb,),
        in_specs=[
            pl.BlockSpec((BN, D), lambda i: (i, 0)),
            pl.BlockSpec((BN, D), lambda i: (i + nb, 0)),
            pl.BlockSpec((BN, D), lambda i: (i, 0)),
            pl.BlockSpec((1, 1), lambda i: (0, 0)),
        ],
        out_specs=pl.BlockSpec((1, 1), lambda i: (0, 0)),
        out_shape=jax.ShapeDtypeStruct((1, 1), jnp.float32),
    )(s_big, s_big, storage, base_pp)


def kernel(x_src, x_dst, edge_feats, last_h_storage, W_enc, b_enc, W_edge,
           edge_index, t, src, dst):
    del last_h_storage, t, dst
    b2 = b_enc.reshape(1, D)
    involved = edge_index.reshape(-1)

    efp = edge_feats.reshape(E // 8, 8 * DE)

    parts = _winner_partial(involved)
    w = _winner_merge(parts)
    fa, fb, efw = _gather_winners(w, x_src, x_dst, efp)
    base_pp, h_src = _j1_call(x_src, x_dst, edge_feats, W_enc, b2, W_edge)
    storage = _j4_call(fa.reshape(PAD_N, D), fb.reshape(PAD_N, D),
                       efw.reshape(PAD_N, D), w.reshape(PAD_N, 1), W_enc, b2, W_edge)
    s_big = _segsum(h_src, src)
    out = _j6_call(s_big, storage, base_pp)
    return out.reshape(1)


# speedup vs baseline: 1.6178x; 1.6178x over previous
import functools

import jax
import jax.numpy as jnp
from jax import lax
from jax.experimental import pallas as pl
from jax.experimental.pallas import tpu as pltpu
from jax.experimental.pallas import tpu_sc as plsc

N_NODES = 10000
D = 128
DE = 16
E = 320000
TWO_E = 2 * E

NC, NS = 2, 16
NW = NC * NS
PAD_N = 10240

_mesh = plsc.VectorSubcoreMesh(core_axis_name="c", subcore_axis_name="s")


def _wid():
    return lax.axis_index("s") * NC + lax.axis_index("c")


CH2 = TWO_E // NW
NCH2 = CH2 // 16


@functools.partial(
    pl.kernel,
    out_type=jax.ShapeDtypeStruct((NW * PAD_N,), jnp.int32),
    mesh=_mesh,
    compiler_params=pltpu.CompilerParams(needs_layout_passes=False),
    scratch_types=[pltpu.VMEM((CH2,), jnp.int32), pltpu.VMEM((PAD_N,), jnp.int32)],
)
def _winner_partial(inv_hbm, parts_hbm, idx_v, wtab):
    wid = _wid()
    pltpu.sync_copy(inv_hbm.at[pl.ds(wid * CH2, CH2)], idx_v)
    neg1 = jnp.full((16,), -1, jnp.int32)

    def initb(i, carry):
        wtab[pl.ds(i * 16, 16)] = neg1
        return carry

    lax.fori_loop(0, PAD_N // 16, initb, 0)

    lanes = lax.iota(jnp.int32, 16)
    shift1 = jnp.minimum(lanes + 1, 15)
    dnums = lax.GatherDimensionNumbers(
        offset_dims=(), collapsed_slice_dims=(0,), start_index_map=(0,))
    last_lane = lanes == 15
    base_j = wid * CH2

    def body(k, carry):
        idx16 = idx_v[pl.ds(k * 16, 16)]
        j16 = base_j + k * 16 + lanes
        key = idx16 * 16 + lanes
        sk, sv = plsc.sort_key_val(key, j16)
        sidx = lax.shift_right_logical(sk, 4)
        nxt = lax.gather(sidx, shift1[:, None], dnums, (1,),
                         mode=lax.GatherScatterMode.PROMISE_IN_BOUNDS)
        winner = (sidx != nxt) | last_lane
        plsc.store_scatter(wtab, [sidx], sv, mask=winner)
        return carry

    lax.fori_loop(0, NCH2, body, 0)
    pltpu.sync_copy(wtab, parts_hbm.at[pl.ds(wid * PAD_N, PAD_N)])


MN = PAD_N // NW


@functools.partial(
    pl.kernel,
    out_type=jax.ShapeDtypeStruct((PAD_N,), jnp.int32),
    mesh=_mesh,
    compiler_params=pltpu.CompilerParams(needs_layout_passes=False),
    scratch_types=[pltpu.VMEM((NW * MN,), jnp.int32), pltpu.VMEM((MN,), jnp.int32)],
)
def _winner_merge(parts_hbm, w_hbm, pv, wout):
    wid = _wid()
    n0 = wid * MN
    for tt in range(NW):
        pltpu.sync_copy(parts_hbm.at[pl.ds(tt * PAD_N + n0, MN)],
                        pv.at[pl.ds(tt * MN, MN)])
    for v in range(MN // 16):
        m = jnp.full((16,), -1, jnp.int32)
        for tt in range(NW):
            m = jnp.maximum(m, pv[pl.ds(tt * MN + v * 16, 16)])
        wout[pl.ds(v * 16, 16)] = m
    pltpu.sync_copy(wout, w_hbm.at[pl.ds(n0, MN)])


RT = PAD_N // NW
NCK, CKS = 4, 80


@functools.partial(
    pl.kernel,
    out_type=(
        jax.ShapeDtypeStruct((PAD_N, D), jnp.float32),
        jax.ShapeDtypeStruct((PAD_N, D), jnp.float32),
    ),
    mesh=_mesh,
    compiler_params=pltpu.CompilerParams(needs_layout_passes=False),
    scratch_types=[
        pltpu.VMEM((RT,), jnp.int32),
        pltpu.VMEM((RT, D), jnp.float32),
        pltpu.VMEM((RT, D), jnp.float32),
        pltpu.VMEM((16, D), jnp.float32),
        pltpu.SemaphoreType.DMA,
        pltpu.SemaphoreType.DMA,
    ],
)
def _gather_winners(w_hbm, xs_hbm, xd_hbm, ef_hbm, f_hbm, efw_hbm,
                    wv, rb, rc, tmp, sem, semf):
    wid = _wid()
    n0 = wid * RT
    pltpu.sync_copy(w_hbm.at[pl.ds(n0, RT)], wv)
    zero = jnp.zeros((16,), jnp.int32)
    copies = []
    for k in range(RT // 16):
        wk = wv[pl.ds(k * 16, 16)]
        src_side = wk < E
        wmod = jnp.maximum(jnp.where(src_side, wk, wk - E), zero)
        copies.append(pltpu.async_copy(
            xd_hbm.at[jnp.where(src_side, zero, wmod)], rb.at[pl.ds(k * 16, 16)], sem))
        copies.append(pltpu.async_copy(
            ef_hbm.at[lax.shift_right_logical(wmod, 3)], rc.at[pl.ds(k * 16, 16)], sem))
    for cp in copies:
        cp.wait()
    pltpu.sync_copy(rb, f_hbm.at[pl.ds(n0, RT)])
    pltpu.sync_copy(rc, efw_hbm.at[pl.ds(n0, RT)])
    for k in range(RT // 16):
        wk = wv[pl.ds(k * 16, 16)]
        fix = (wk < E) & (wk >= 0)
        wmod = jnp.maximum(wk, zero)
        fixi = fix.astype(jnp.int32)

        @pl.when(jnp.max(fixi, axis=0) > 0)
        def _(k=k, fix=fix, fixi=fixi, wmod=wmod):
            pltpu.async_copy(xs_hbm.at[jnp.where(fix, wmod, zero)], tmp, semf).wait()
            for l in range(16):
                @pl.when(fixi[l] > 0)
                def _(k=k, l=l):
                    pltpu.sync_copy(tmp.at[pl.ds(l, 1)], f_hbm.at[pl.ds(n0 + k * 16 + l, 1)])


EPT = E // NW
HB = 80
NHC = EPT // HB
APC = HB // 16
NPS = PAD_N // NS


@functools.partial(
    pl.kernel,
    out_type=jax.ShapeDtypeStruct((2 * PAD_N, D), jnp.float32),
    mesh=_mesh,
    compiler_params=pltpu.CompilerParams(needs_layout_passes=False),
    scratch_types=[
        pltpu.VMEM((EPT,), jnp.int32),
        pltpu.VMEM((HB, D), jnp.float32),
        pltpu.VMEM((HB, D), jnp.float32),
        pltpu.VMEM_SHARED((PAD_N, D), jnp.float32),
        pltpu.SemaphoreType.DMA,
        pltpu.SemaphoreType.DMA,
        pltpu.SemaphoreType.DMA,
    ],
)
def _segsum(h_hbm, src_hbm, s_hbm, idx_all, st0, st1, s_sh, semh0, semh1, sema):
    core = lax.axis_index("c")
    sid = lax.axis_index("s")
    wid = sid * NC + core
    e_base = wid * EPT
    sts = (st0, st1)
    semhs = (semh0, semh1)
    pltpu.sync_copy(src_hbm.at[pl.ds(e_base, EPT)], idx_all)

    zero16 = jnp.zeros((16,), jnp.float32)

    def zrow(r, carry):
        for q in range(D // 16):
            st0[r, pl.ds(q * 16, 16)] = zero16
        return carry

    lax.fori_loop(0, HB, zrow, 0)
    for q in range(NPS // HB):
        pltpu.sync_copy(st0, s_sh.at[pl.ds(sid * NPS + q * HB, HB)])
    plsc.subcore_barrier()

    def h_chunk_copy(c, b):
        return pltpu.async_copy(h_hbm.at[pl.ds(e_base + c * HB, HB)], sts[b], semhs[b])

    h_chunk_copy(0, 0)
    h_chunk_copy(1, 1)

    def do_chunk(c, b):
        pltpu.make_async_copy(h_hbm.at[pl.ds(e_base + c * HB, HB)], sts[b], semhs[b]).wait()
        descs = []
        for u in range(APC):
            idx16 = idx_all[pl.ds(c * HB + u * 16, 16)]
            descs.append(pltpu.async_copy(
                sts[b].at[pl.ds(u * 16, 16)], s_sh.at[idx16], sema, add=True))
        for dd in descs:
            dd.wait()

    def pair(g, carry):
        for b in range(2):
            c = 2 * g + b
            do_chunk(c, b)

            @pl.when(c + 2 < NHC)
            def _():
                h_chunk_copy(c + 2, b)
        return carry

    lax.fori_loop(0, NHC // 2, pair, 0)
    if NHC % 2:
        do_chunk(NHC - 1, 0)
    plsc.subcore_barrier()
    row0 = core * PAD_N + sid * NPS
    for q in range(NPS // HB):
        pltpu.sync_copy(s_sh.at[pl.ds(sid * NPS + q * HB, HB)],
                        s_hbm.at[pl.ds(row0 + q * HB, HB)])


BE = 2000


def _j1_body(xs_ref, xd_ref, ef_ref, wn_ref, b_ref, we_ref, base_ref, h_ref):
    ep = jnp.dot(ef_ref[...], we_ref[...], preferred_element_type=jnp.float32) + b_ref[...]
    hs = jnp.maximum(jnp.dot(xs_ref[...], wn_ref[...], preferred_element_type=jnp.float32) + ep, 0.0)
    hd = jnp.maximum(jnp.dot(xd_ref[...], wn_ref[...], preferred_element_type=jnp.float32) + ep, 0.0)
    h_ref[...] = hs
    d = jnp.sum(hs * hd, axis=1)
    part = jnp.sum(d * d).reshape(1, 1)

    @pl.when(pl.program_id(0) == 0)
    def _():
        base_ref[...] = part

    @pl.when(pl.program_id(0) != 0)
    def _():
        base_ref[...] = base_ref[...] + part


def _j1_call(x_src, x_dst, edge_feats, W_enc, b2, W_edge):
    return pl.pallas_call(
        _j1_body,
        grid=(E // BE,),
        in_specs=[
            pl.BlockSpec((BE, D), lambda i: (i, 0)),
            pl.BlockSpec((BE, D), lambda i: (i, 0)),
            pl.BlockSpec((BE, DE), lambda i: (i, 0)),
            pl.BlockSpec((D, D), lambda i: (0, 0)),
            pl.BlockSpec((1, D), lambda i: (0, 0)),
            pl.BlockSpec((DE, D), lambda i: (0, 0)),
        ],
        out_specs=[
            pl.BlockSpec((1, 1), lambda i: (0, 0)),
            pl.BlockSpec((BE, D), lambda i: (i, 0)),
        ],
        out_shape=[
            jax.ShapeDtypeStruct((1, 1), jnp.float32),
            jax.ShapeDtypeStruct((E, D), jnp.float32),
        ],
    )(x_src, x_dst, edge_feats, W_enc, b2, W_edge)


BN = 256


def _j4_body(fa_ref, ef_ref, w_ref, wn_ref, b_ref, we_ref, st_ref):
    w = w_ref[...]
    feats = fa_ref[...]
    wmod = jnp.maximum(jnp.where(w < E, w, w - E), 0)
    p = jnp.bitwise_and(wmod, 7)
    efp = ef_ref[...]
    ef16 = jnp.zeros((BN, DE), jnp.float32)
    for pp in range(8):
        ef16 = ef16 + jnp.where(p == pp, efp[:, pp * DE:(pp + 1) * DE], 0.0)
    h = jnp.dot(feats, wn_ref[...], preferred_element_type=jnp.float32)
    h = h + jnp.dot(ef16, we_ref[...], preferred_element_type=jnp.float32) + b_ref[...]
    h = jnp.maximum(h, 0.0)
    st_ref[...] = jnp.where(w >= 0, h, 0.0)


def _j4_call(fa, efw, w2, W_enc, b2, W_edge):
    return pl.pallas_call(
        _j4_body,
        grid=(PAD_N // BN,),
        in_specs=[
            pl.BlockSpec((BN, D), lambda i: (i, 0)),
            pl.BlockSpec((BN, D), lambda i: (i, 0)),
            pl.BlockSpec((BN, 1), lambda i: (i, 0)),
            pl.BlockSpec((D, D), lambda i: (0, 0)),
            pl.BlockSpec((1, D), lambda i: (0, 0)),
            pl.BlockSpec((DE, D), lambda i: (0, 0)),
        ],
        out_specs=pl.BlockSpec((BN, D), lambda i: (i, 0)),
        out_shape=jax.ShapeDtypeStruct((PAD_N, D), jnp.float32),
    )(fa, efw, w2, W_enc, b2, W_edge)


def _j6_body(s0_ref, s1_ref, st_ref, base_ref, o_ref):
    inv_e = 1.0 / E
    val = jnp.sum((s0_ref[...] + s1_ref[...]) * st_ref[...]).reshape(1, 1)

    @pl.when(pl.program_id(0) == 0)
    def _():
        o_ref[...] = base_ref[...] * inv_e + 0.001 * inv_e * val

    @pl.when(pl.program_id(0) != 0)
    def _():
        o_ref[...] = o_ref[...] + 0.001 * inv_e * val


def _j6_call(s_big, storage, base_pp):
    nb = PAD_N // BN
    return pl.pallas_call(
        _j6_body,
        grid=(nb,),
        in_specs=[
            pl.BlockSpec((BN, D), lambda i: (i, 0)),
            pl.BlockSpec((BN, D), lambda i: (i + nb, 0)),
            pl.BlockSpec((BN, D), lambda i: (i, 0)),
            pl.BlockSpec((1, 1), lambda i: (0, 0)),
        ],
        out_specs=pl.BlockSpec((1, 1), lambda i: (0, 0)),
        out_shape=jax.ShapeDtypeStruct((1, 1), jnp.float32),
    )(s_big, s_big, storage, base_pp)


def kernel(x_src, x_dst, edge_feats, last_h_storage, W_enc, b_enc, W_edge,
           edge_index, t, src, dst):
    del last_h_storage, t, dst
    b2 = b_enc.reshape(1, D)
    involved = edge_index.reshape(-1)

    efp = edge_feats.reshape(E // 8, 8 * DE)

    parts = _winner_partial(involved)
    w = _winner_merge(parts)
    fa, efw = _gather_winners(w, x_src, x_dst, efp)
    base_pp, h_src = _j1_call(x_src, x_dst, edge_feats, W_enc, b2, W_edge)
    storage = _j4_call(fa, efw, w.reshape(PAD_N, 1), W_enc, b2, W_edge)
    s_big = _segsum(h_src, src)
    out = _j6_call(s_big, storage, base_pp)
    return out.reshape(1)


# speedup vs baseline: 1.7548x; 1.0847x over previous
import functools

import jax
import jax.numpy as jnp
from jax import lax
from jax.experimental import pallas as pl
from jax.experimental.pallas import tpu as pltpu
from jax.experimental.pallas import tpu_sc as plsc

N_NODES = 10000
D = 128
DE = 16
E = 320000
TWO_E = 2 * E

NC, NS = 2, 16
NW = NC * NS
PAD_N = 10240

_mesh = plsc.VectorSubcoreMesh(core_axis_name="c", subcore_axis_name="s")


def _wid():
    return lax.axis_index("s") * NC + lax.axis_index("c")


CH2 = TWO_E // NW
NCH2 = CH2 // 16


@functools.partial(
    pl.kernel,
    out_type=jax.ShapeDtypeStruct((NW * PAD_N,), jnp.int32),
    mesh=_mesh,
    compiler_params=pltpu.CompilerParams(needs_layout_passes=False),
    scratch_types=[pltpu.VMEM((CH2,), jnp.int32), pltpu.VMEM((PAD_N,), jnp.int32)],
)
def _winner_partial(inv_hbm, parts_hbm, idx_v, wtab):
    wid = _wid()
    pltpu.sync_copy(inv_hbm.at[pl.ds(wid * CH2, CH2)], idx_v)
    neg1 = jnp.full((16,), -1, jnp.int32)

    def initb(i, carry):
        wtab[pl.ds(i * 16, 16)] = neg1
        return carry

    lax.fori_loop(0, PAD_N // 16, initb, 0)

    lanes = lax.iota(jnp.int32, 16)
    shift1 = jnp.minimum(lanes + 1, 15)
    dnums = lax.GatherDimensionNumbers(
        offset_dims=(), collapsed_slice_dims=(0,), start_index_map=(0,))
    last_lane = lanes == 15
    base_j = wid * CH2

    def body(k, carry):
        idx16 = idx_v[pl.ds(k * 16, 16)]
        j16 = base_j + k * 16 + lanes
        key = idx16 * 16 + lanes
        sk, sv = plsc.sort_key_val(key, j16)
        sidx = lax.shift_right_logical(sk, 4)
        nxt = lax.gather(sidx, shift1[:, None], dnums, (1,),
                         mode=lax.GatherScatterMode.PROMISE_IN_BOUNDS)
        winner = (sidx != nxt) | last_lane
        plsc.store_scatter(wtab, [sidx], sv, mask=winner)
        return carry

    lax.fori_loop(0, NCH2, body, 0)
    pltpu.sync_copy(wtab, parts_hbm.at[pl.ds(wid * PAD_N, PAD_N)])


MN = PAD_N // NW


@functools.partial(
    pl.kernel,
    out_type=jax.ShapeDtypeStruct((PAD_N,), jnp.int32),
    mesh=_mesh,
    compiler_params=pltpu.CompilerParams(needs_layout_passes=False),
    scratch_types=[pltpu.VMEM((NW * MN,), jnp.int32), pltpu.VMEM((MN,), jnp.int32)],
)
def _winner_merge(parts_hbm, w_hbm, pv, wout):
    wid = _wid()
    n0 = wid * MN
    for tt in range(NW):
        pltpu.sync_copy(parts_hbm.at[pl.ds(tt * PAD_N + n0, MN)],
                        pv.at[pl.ds(tt * MN, MN)])
    for v in range(MN // 16):
        m = jnp.full((16,), -1, jnp.int32)
        for tt in range(NW):
            m = jnp.maximum(m, pv[pl.ds(tt * MN + v * 16, 16)])
        wout[pl.ds(v * 16, 16)] = m
    pltpu.sync_copy(wout, w_hbm.at[pl.ds(n0, MN)])


RT = PAD_N // NW
NCK, CKS = 4, 80


@functools.partial(
    pl.kernel,
    out_type=(
        jax.ShapeDtypeStruct((PAD_N, D), jnp.float32),
        jax.ShapeDtypeStruct((PAD_N, D), jnp.float32),
    ),
    mesh=_mesh,
    compiler_params=pltpu.CompilerParams(needs_layout_passes=False),
    scratch_types=[
        pltpu.VMEM((RT,), jnp.int32),
        pltpu.VMEM((RT, D), jnp.float32),
        pltpu.VMEM((RT, D), jnp.float32),
        pltpu.VMEM((16, D), jnp.float32),
        pltpu.SemaphoreType.DMA,
        pltpu.SemaphoreType.DMA,
    ],
)
def _gather_winners(w_hbm, xs_hbm, xd_hbm, ef_hbm, f_hbm, efw_hbm,
                    wv, rb, rc, tmp, sem, semf):
    wid = _wid()
    n0 = wid * RT
    pltpu.sync_copy(w_hbm.at[pl.ds(n0, RT)], wv)
    zero = jnp.zeros((16,), jnp.int32)
    copies = []
    for k in range(RT // 16):
        wk = wv[pl.ds(k * 16, 16)]
        src_side = wk < E
        wmod = jnp.maximum(jnp.where(src_side, wk, wk - E), zero)
        copies.append(pltpu.async_copy(
            xd_hbm.at[jnp.where(src_side, zero, wmod)], rb.at[pl.ds(k * 16, 16)], sem))
        copies.append(pltpu.async_copy(
            ef_hbm.at[lax.shift_right_logical(wmod, 3)], rc.at[pl.ds(k * 16, 16)], sem))
    for cp in copies:
        cp.wait()
    pltpu.sync_copy(rb, f_hbm.at[pl.ds(n0, RT)])
    pltpu.sync_copy(rc, efw_hbm.at[pl.ds(n0, RT)])
    for k in range(RT // 16):
        wk = wv[pl.ds(k * 16, 16)]
        fix = (wk < E) & (wk >= 0)
        wmod = jnp.maximum(wk, zero)
        fixi = fix.astype(jnp.int32)

        @pl.when(jnp.max(fixi, axis=0) > 0)
        def _(k=k, fix=fix, fixi=fixi, wmod=wmod):
            pltpu.async_copy(xs_hbm.at[jnp.where(fix, wmod, zero)], tmp, semf).wait()
            for l in range(16):
                @pl.when(fixi[l] > 0)
                def _(k=k, l=l):
                    pltpu.sync_copy(tmp.at[pl.ds(l, 1)], f_hbm.at[pl.ds(n0 + k * 16 + l, 1)])


EPT = E // NW
HB = 80
NHC = EPT // HB
APC = HB // 16
NPS = PAD_N // NS


@functools.partial(
    pl.kernel,
    out_type=jax.ShapeDtypeStruct((2 * PAD_N, D), jnp.float32),
    mesh=_mesh,
    compiler_params=pltpu.CompilerParams(needs_layout_passes=False),
    scratch_types=[
        pltpu.VMEM((EPT,), jnp.int32),
        pltpu.VMEM((HB, D), jnp.float32),
        pltpu.VMEM((HB, D), jnp.float32),
        pltpu.VMEM_SHARED((PAD_N, D), jnp.float32),
        pltpu.SemaphoreType.DMA,
        pltpu.SemaphoreType.DMA,
        pltpu.SemaphoreType.DMA,
    ],
)
def _segsum(h_hbm, src_hbm, s_hbm, idx_all, st0, st1, s_sh, semh0, semh1, sema):
    core = lax.axis_index("c")
    sid = lax.axis_index("s")
    wid = sid * NC + core
    e_base = wid * EPT
    sts = (st0, st1)
    semhs = (semh0, semh1)
    pltpu.sync_copy(src_hbm.at[pl.ds(e_base, EPT)], idx_all)

    zero16 = jnp.zeros((16,), jnp.float32)

    def zrow(r, carry):
        for q in range(D // 16):
            st0[r, pl.ds(q * 16, 16)] = zero16
        return carry

    lax.fori_loop(0, HB, zrow, 0)
    for q in range(NPS // HB):
        pltpu.sync_copy(st0, s_sh.at[pl.ds(sid * NPS + q * HB, HB)])
    plsc.subcore_barrier()

    def h_chunk_copy(c, b):
        return pltpu.async_copy(h_hbm.at[pl.ds(e_base + c * HB, HB)], sts[b], semhs[b])

    h_chunk_copy(0, 0)
    h_chunk_copy(1, 1)

    def do_chunk(c, b):
        pltpu.make_async_copy(h_hbm.at[pl.ds(e_base + c * HB, HB)], sts[b], semhs[b]).wait()
        descs = []
        for u in range(APC):
            idx16 = idx_all[pl.ds(c * HB + u * 16, 16)]
            descs.append(pltpu.async_copy(
                sts[b].at[pl.ds(u * 16, 16)], s_sh.at[idx16], sema, add=True))
        for dd in descs:
            dd.wait()

    def pair(g, carry):
        for b in range(2):
            c = 2 * g + b
            do_chunk(c, b)

            @pl.when(c + 2 < NHC)
            def _():
                h_chunk_copy(c + 2, b)
        return carry

    lax.fori_loop(0, NHC // 2, pair, 0)
    if NHC % 2:
        do_chunk(NHC - 1, 0)
    plsc.subcore_barrier()
    row0 = core * PAD_N + sid * NPS
    for q in range(NPS // HB):
        pltpu.sync_copy(s_sh.at[pl.ds(sid * NPS + q * HB, HB)],
                        s_hbm.at[pl.ds(row0 + q * HB, HB)])


BE = 4000


def _j1_body(xs_ref, xd_ref, ef_ref, wn_ref, b_ref, we_ref, base_ref, h_ref):
    ep = jnp.dot(ef_ref[...], we_ref[...], preferred_element_type=jnp.float32) + b_ref[...]
    hs = jnp.maximum(jnp.dot(xs_ref[...], wn_ref[...], preferred_element_type=jnp.float32) + ep, 0.0)
    hd = jnp.maximum(jnp.dot(xd_ref[...], wn_ref[...], preferred_element_type=jnp.float32) + ep, 0.0)
    h_ref[...] = hs
    d = jnp.sum(hs * hd, axis=1)
    part = jnp.sum(d * d).reshape(1, 1)

    @pl.when(pl.program_id(0) == 0)
    def _():
        base_ref[...] = part

    @pl.when(pl.program_id(0) != 0)
    def _():
        base_ref[...] = base_ref[...] + part


def _j1_call(x_src, x_dst, edge_feats, W_enc, b2, W_edge):
    return pl.pallas_call(
        _j1_body,
        grid=(E // BE,),
        in_specs=[
            pl.BlockSpec((BE, D), lambda i: (i, 0)),
            pl.BlockSpec((BE, D), lambda i: (i, 0)),
            pl.BlockSpec((BE, DE), lambda i: (i, 0)),
            pl.BlockSpec((D, D), lambda i: (0, 0)),
            pl.BlockSpec((1, D), lambda i: (0, 0)),
            pl.BlockSpec((DE, D), lambda i: (0, 0)),
        ],
        out_specs=[
            pl.BlockSpec((1, 1), lambda i: (0, 0)),
            pl.BlockSpec((BE, D), lambda i: (i, 0)),
        ],
        out_shape=[
            jax.ShapeDtypeStruct((1, 1), jnp.float32),
            jax.ShapeDtypeStruct((E, D), jnp.float32),
        ],
    )(x_src, x_dst, edge_feats, W_enc, b2, W_edge)


BN = 256


def _j4_body(fa_ref, ef_ref, w_ref, wn_ref, b_ref, we_ref, st_ref):
    w = w_ref[...]
    feats = fa_ref[...]
    wmod = jnp.maximum(jnp.where(w < E, w, w - E), 0)
    p = jnp.bitwise_and(wmod, 7)
    efp = ef_ref[...]
    ef16 = jnp.zeros((BN, DE), jnp.float32)
    for pp in range(8):
        ef16 = ef16 + jnp.where(p == pp, efp[:, pp * DE:(pp + 1) * DE], 0.0)
    h = jnp.dot(feats, wn_ref[...], preferred_element_type=jnp.float32)
    h = h + jnp.dot(ef16, we_ref[...], preferred_element_type=jnp.float32) + b_ref[...]
    h = jnp.maximum(h, 0.0)
    st_ref[...] = jnp.where(w >= 0, h, 0.0)


def _j4_call(fa, efw, w2, W_enc, b2, W_edge):
    return pl.pallas_call(
        _j4_body,
        grid=(PAD_N // BN,),
        in_specs=[
            pl.BlockSpec((BN, D), lambda i: (i, 0)),
            pl.BlockSpec((BN, D), lambda i: (i, 0)),
            pl.BlockSpec((BN, 1), lambda i: (i, 0)),
            pl.BlockSpec((D, D), lambda i: (0, 0)),
            pl.BlockSpec((1, D), lambda i: (0, 0)),
            pl.BlockSpec((DE, D), lambda i: (0, 0)),
        ],
        out_specs=pl.BlockSpec((BN, D), lambda i: (i, 0)),
        out_shape=jax.ShapeDtypeStruct((PAD_N, D), jnp.float32),
    )(fa, efw, w2, W_enc, b2, W_edge)


def _j6_body(s0_ref, s1_ref, st_ref, base_ref, o_ref):
    inv_e = 1.0 / E
    val = jnp.sum((s0_ref[...] + s1_ref[...]) * st_ref[...]).reshape(1, 1)

    @pl.when(pl.program_id(0) == 0)
    def _():
        o_ref[...] = base_ref[...] * inv_e + 0.001 * inv_e * val

    @pl.when(pl.program_id(0) != 0)
    def _():
        o_ref[...] = o_ref[...] + 0.001 * inv_e * val


def _j6_call(s_big, storage, base_pp):
    nb = PAD_N // BN
    return pl.pallas_call(
        _j6_body,
        grid=(nb,),
        in_specs=[
            pl.BlockSpec((BN, D), lambda i: (i, 0)),
            pl.BlockSpec((BN, D), lambda i: (i + nb, 0)),
            pl.BlockSpec((BN, D), lambda i: (i, 0)),
            pl.BlockSpec((1, 1), lambda i: (0, 0)),
        ],
        out_specs=pl.BlockSpec((1, 1), lambda i: (0, 0)),
        out_shape=jax.ShapeDtypeStruct((1, 1), jnp.float32),
    )(s_big, s_big, storage, base_pp)


def kernel(x_src, x_dst, edge_feats, last_h_storage, W_enc, b_enc, W_edge,
           edge_index, t, src, dst):
    del last_h_storage, t, dst
    b2 = b_enc.reshape(1, D)
    involved = edge_index.reshape(-1)

    efp = edge_feats.reshape(E // 8, 8 * DE)

    parts = _winner_partial(involved)
    w = _winner_merge(parts)
    fa, efw = _gather_winners(w, x_src, x_dst, efp)
    base_pp, h_src = _j1_call(x_src, x_dst, edge_feats, W_enc, b2, W_edge)
    storage = _j4_call(fa, efw, w.reshape(PAD_N, 1), W_enc, b2, W_edge)
    s_big = _segsum(h_src, src)
    out = _j6_call(s_big, storage, base_pp)
    return out.reshape(1)
